# baseline (speedup 1.0000x reference)
import functools

import jax
import jax.numpy as jnp
from jax import lax
from jax.experimental import pallas as pl
from jax.experimental.pallas import tpu as pltpu
from jax.experimental.pallas import tpu_sc as plsc

_DIM = 64
_L = 16
_NW = 32
_BATCH = 16384
_MARGIN = 1.0
_PER_W = _BATCH // _NW
_CH = 128
_NCH = _PER_W // _CH
_GC = _CH // _L


def _rsqrt(x):
    xi = lax.bitcast_convert_type(x, jnp.int32)
    yi = jnp.int32(0x5F3759DF) - (xi >> 1)
    y = lax.bitcast_convert_type(yi, jnp.float32)
    for _ in range(3):
        y = y * (1.5 - 0.5 * x * y * y)
    return y


def _sc_body(ent_hbm, rel_hbm, heads_hbm, rels_hbm, tails_hbm, out_hbm,
             hidx_v, ridx_v, tidx_v, hrows, rrows, trows,
             epos, eneg, ostage, sem):
    wid = lax.axis_index("s") * 2 + lax.axis_index("c")
    iota = lax.iota(jnp.int32, _L)

    for pol in range(2):
        eref = epos if pol == 0 else eneg
        for c in range(_NCH):
            base = pol * _BATCH + wid * _PER_W + c * _CH
            pltpu.sync_copy(heads_hbm.at[pl.ds(base, _CH)], hidx_v)
            pltpu.sync_copy(rels_hbm.at[pl.ds(base, _CH)], ridx_v)
            pltpu.sync_copy(tails_hbm.at[pl.ds(base, _CH)], tidx_v)
            cp1 = pltpu.async_copy(ent_hbm.at[hidx_v], hrows, sem)
            cp2 = pltpu.async_copy(rel_hbm.at[ridx_v], rrows, sem)
            cp3 = pltpu.async_copy(ent_hbm.at[tidx_v], trows, sem)
            cp1.wait()
            cp2.wait()
            cp3.wait()

            for g in range(_GC):
                lanes = g * _L + iota

                def d_body(d, carry):
                    hh, rr, tt, hr, ht, rt = carry
                    dv = jnp.full((_L,), 0, jnp.int32) + d
                    hv = plsc.load_gather(hrows, [lanes, dv])
                    rv = plsc.load_gather(rrows, [lanes, dv])
                    tv = plsc.load_gather(trows, [lanes, dv])
                    return (hh + hv * hv, rr + rv * rv, tt + tv * tv,
                            hr + hv * rv, ht + hv * tv, rt + rv * tv)

                z = jnp.zeros((_L,), jnp.float32)
                hh, rr, tt, hr, ht, rt = lax.fori_loop(0, _DIM, d_body,
                                                       (z, z, z, z, z, z))
                ih = _rsqrt(jnp.maximum(hh, 1e-24))
                ir = _rsqrt(jnp.maximum(rr, 1e-24))
                it = _rsqrt(jnp.maximum(tt, 1e-24))
                e2 = (hh * ih * ih + rr * ir * ir + tt * it * it
                      + 2.0 * (hr * (ih * ir) - ht * (ih * it)
                               - rt * (ir * it)))
                e2 = jnp.maximum(e2, 0.0)
                e = e2 * _rsqrt(jnp.maximum(e2, 1e-30))
                eref[pl.ds(c * _CH + g * _L, _L)] = e

    acc = jnp.zeros((_L,), jnp.float32)
    for g in range(_PER_W // _L):
        lp = epos[pl.ds(g * _L, _L)]
        ln = eneg[pl.ds(g * _L, _L)]
        acc = acc + jnp.maximum(_MARGIN + lp - ln, 0.0)
    ostage[...] = acc
    pltpu.sync_copy(ostage, out_hbm.at[wid])


_sc_call = functools.partial(
    pl.kernel,
    mesh=plsc.VectorSubcoreMesh(core_axis_name="c", subcore_axis_name="s"),
    out_type=jax.ShapeDtypeStruct((_NW, _L), jnp.float32),
    scratch_types=[
        pltpu.VMEM((_CH,), jnp.int32),
        pltpu.VMEM((_CH,), jnp.int32),
        pltpu.VMEM((_CH,), jnp.int32),
        pltpu.VMEM((_CH, _DIM), jnp.float32),
        pltpu.VMEM((_CH, _DIM), jnp.float32),
        pltpu.VMEM((_CH, _DIM), jnp.float32),
        pltpu.VMEM((_PER_W,), jnp.float32),
        pltpu.VMEM((_PER_W,), jnp.float32),
        pltpu.VMEM((_L,), jnp.float32),
        pltpu.SemaphoreType.DMA,
    ],
    compiler_params=pltpu.CompilerParams(needs_layout_passes=False,
                                         use_tc_tiling_on_sc=False),
)(_sc_body)


def kernel(pos_triples, neg_triples, ent_emb, rel_emb):
    tri = jnp.concatenate([pos_triples, neg_triples], axis=0).astype(jnp.int32)
    heads = tri[:, 0]
    rels = tri[:, 1]
    tails = tri[:, 2]
    partials = _sc_call(ent_emb, rel_emb, heads, rels, tails)
    return jnp.sum(partials) / jnp.float32(_BATCH)
